# baseline (speedup 1.0000x reference)
import functools

import jax
import jax.numpy as jnp
from jax import lax
from jax.experimental import pallas as pl
from jax.experimental.pallas import tpu as pltpu
from jax.experimental.pallas import tpu_sc as plsc

N = 10000
E = 320000
D = 128
NC = 2
NS = 16
NW = NC * NS
CHUNK = 128
RING = 2
ISLOTS = 4
EPW = E // NW
F = EPW // CHUNK
TAIL = EPW - F * CHUNK
MAIN = F - 6
N_PAD = 10240
RPT = N_PAD // NS

_mesh = plsc.VectorSubcoreMesh(core_axis_name="core", subcore_axis_name="subcore")

f32 = jnp.float32


@functools.partial(
    pl.kernel,
    out_type=jax.ShapeDtypeStruct((NC, N_PAD), f32),
    mesh=_mesh,
    scratch_types=[
        pltpu.VMEM((ISLOTS, CHUNK), jnp.int32),
        pltpu.VMEM((TAIL,), jnp.int32),
        pltpu.VMEM((CHUNK,), f32),
        pltpu.VMEM_SHARED((N_PAD,), f32),
        pltpu.SemaphoreType.DMA((ISLOTS,)),
    ],
)
def _deg_kernel(dst_hbm, zeros_hbm, ones_hbm, deg_hbm,
                dst_v, tdst_v, ones_v, acc, dsems):
    cid = lax.axis_index("core")
    sid = lax.axis_index("subcore")
    wid = cid * NS + sid
    base = wid * EPW
    row0 = sid * RPT

    def _idst(k, j):
        return pltpu.make_async_copy(dst_hbm.at[pl.ds(base + k * CHUNK, CHUNK)],
                                     dst_v.at[j], dsems.at[j])

    for j in range(ISLOTS):
        _idst(j, j).start()
    pltpu.sync_copy(zeros_hbm.at[pl.ds(row0, RPT)], acc.at[pl.ds(row0, RPT)])
    pltpu.sync_copy(ones_hbm, ones_v)
    plsc.subcore_barrier()

    @pl.loop(0, MAIN, step=ISLOTS)
    def _(kbase):
        for jj in range(ISLOTS):
            k = kbase + jj
            _idst(k, jj).wait()
            pltpu.sync_copy(ones_v, acc.at[dst_v.at[jj]], add=True)
            _idst(k + ISLOTS, jj).start()

    for k in range(MAIN, F):
        jj = k % ISLOTS
        _idst(k, jj).wait()
        pltpu.sync_copy(ones_v, acc.at[dst_v.at[jj]], add=True)
        if k + ISLOTS < F:
            _idst(k + ISLOTS, jj).start()

    pltpu.sync_copy(dst_hbm.at[pl.ds(base + F * CHUNK, TAIL)], tdst_v)
    pltpu.sync_copy(ones_v.at[pl.ds(0, TAIL)], acc.at[tdst_v], add=True)

    plsc.subcore_barrier()
    pltpu.sync_copy(acc.at[pl.ds(row0, RPT)], deg_hbm.at[cid, pl.ds(row0, RPT)])


@functools.partial(
    pl.kernel,
    out_type=jax.ShapeDtypeStruct((NC, N_PAD, D), f32),
    mesh=_mesh,
    scratch_types=[
        pltpu.VMEM((ISLOTS, CHUNK), jnp.int32),
        pltpu.VMEM((ISLOTS, CHUNK), jnp.int32),
        pltpu.VMEM((TAIL,), jnp.int32),
        pltpu.VMEM((TAIL,), jnp.int32),
        pltpu.VMEM((RING, CHUNK, D), f32),
        pltpu.VMEM_SHARED((N_PAD, D), f32),
        pltpu.SemaphoreType.DMA((RING,)),
        pltpu.SemaphoreType.DMA((ISLOTS,)),
        pltpu.SemaphoreType.DMA((ISLOTS,)),
    ],
)
def _agg_kernel(g_hbm, src_hbm, dst_hbm, zeros_hbm, s_hbm,
                src_v, dst_v, tsrc_v, tdst_v, rows_v, acc, gsems, ssems, dsems):
    cid = lax.axis_index("core")
    sid = lax.axis_index("subcore")
    wid = cid * NS + sid
    base = wid * EPW
    row0 = sid * RPT

    def _isrc(k, j):
        return pltpu.make_async_copy(src_hbm.at[pl.ds(base + k * CHUNK, CHUNK)],
                                     src_v.at[j], ssems.at[j])

    def _idst(k, j):
        return pltpu.make_async_copy(dst_hbm.at[pl.ds(base + k * CHUNK, CHUNK)],
                                     dst_v.at[j], dsems.at[j])

    def _gather(j, b):
        return pltpu.make_async_copy(g_hbm.at[src_v.at[j]], rows_v.at[b],
                                     gsems.at[b])

    for j in range(ISLOTS):
        _isrc(j, j).start()
        _idst(j, j).start()
    pltpu.sync_copy(zeros_hbm.at[pl.ds(row0, RPT)], acc.at[pl.ds(row0, RPT)])
    plsc.subcore_barrier()

    for b in range(RING):
        _isrc(b, b).wait()
        _gather(b, b).start()

    @pl.loop(0, MAIN, step=ISLOTS)
    def _(kbase):
        for jj in range(ISLOTS):
            k = kbase + jj
            b = jj % RING
            _gather(jj, b).wait()
            _idst(k, jj).wait()
            pltpu.sync_copy(rows_v.at[b], acc.at[dst_v.at[jj]], add=True)
            _isrc(k + ISLOTS, jj).start()
            _idst(k + ISLOTS, jj).start()
            j2 = (jj + RING) % ISLOTS
            _isrc(k + RING, j2).wait()
            _gather(j2, b).start()

    for k in range(MAIN, F):
        jj = k % ISLOTS
        b = k % RING
        _gather(jj, b).wait()
        _idst(k, jj).wait()
        pltpu.sync_copy(rows_v.at[b], acc.at[dst_v.at[jj]], add=True)
        if k + ISLOTS < F:
            _isrc(k + ISLOTS, jj).start()
            _idst(k + ISLOTS, jj).start()
        if k + RING < F:
            j2 = (jj + RING) % ISLOTS
            _isrc(k + RING, j2).wait()
            _gather(j2, b).start()

    pltpu.sync_copy(src_hbm.at[pl.ds(base + F * CHUNK, TAIL)], tsrc_v)
    pltpu.sync_copy(dst_hbm.at[pl.ds(base + F * CHUNK, TAIL)], tdst_v)
    pltpu.sync_copy(g_hbm.at[tsrc_v], rows_v.at[0, pl.ds(0, TAIL)])
    pltpu.sync_copy(rows_v.at[0, pl.ds(0, TAIL)], acc.at[tdst_v], add=True)

    plsc.subcore_barrier()
    pltpu.sync_copy(acc.at[pl.ds(row0, RPT)], s_hbm.at[cid, pl.ds(row0, RPT)])


BR = 512
GRID = N_PAD // BR
_HI = lax.Precision.HIGHEST


def _mm(a, w):
    return jnp.dot(a, w, preferred_element_type=f32, precision=_HI)


def _k1_body(x_ref, win_ref, bin_ref, w1_ref, b1_ref, d0_ref, d1_ref,
             g_ref, dinv_ref):
    dinv = lax.rsqrt(d0_ref[...] + d1_ref[...] + 1.0)
    h = jnp.maximum(_mm(x_ref[...], win_ref[...]) + bin_ref[...], 0.0)
    g_ref[...] = (_mm(h, w1_ref[...]) + b1_ref[...]) * dinv
    dinv_ref[...] = dinv


def _layer_body(s_ref, g_ref, w_ref, b_ref, dinv_ref, out_ref):
    dinv = dinv_ref[...]
    x = jnp.maximum((s_ref[0] + s_ref[1] + g_ref[...]) * dinv, 0.0)
    out_ref[...] = (_mm(x, w_ref[...]) + b_ref[...]) * dinv


def _head_body(s_ref, g_ref, dinv_ref, wout_ref, bout_ref, out_ref):
    x = jnp.maximum((s_ref[0] + s_ref[1] + g_ref[...]) * dinv_ref[...], 0.0)
    out_ref[...] = _mm(x, wout_ref[...]) + bout_ref[...]


_full2 = lambda shape: pl.BlockSpec(shape, lambda i: (0, 0))
_rows = lambda w: pl.BlockSpec((BR, w), lambda i: (i, 0))
_srow = pl.BlockSpec((2, BR, D), lambda i: (0, i, 0))

_k1_call = pl.pallas_call(
    _k1_body,
    grid=(GRID,),
    in_specs=[_rows(D), _full2((D, D)), _full2((1, D)), _full2((D, D)),
              _full2((1, D)), _rows(1), _rows(1)],
    out_specs=[_rows(D), _rows(1)],
    out_shape=[jax.ShapeDtypeStruct((N_PAD, D), f32),
               jax.ShapeDtypeStruct((N_PAD, 1), f32)],
)

_layer_call = pl.pallas_call(
    _layer_body,
    grid=(GRID,),
    in_specs=[_srow, _rows(D), _full2((D, D)), _full2((1, D)), _rows(1)],
    out_specs=_rows(D),
    out_shape=jax.ShapeDtypeStruct((N_PAD, D), f32),
)

_head_call = pl.pallas_call(
    _head_body,
    grid=(GRID,),
    in_specs=[_srow, _rows(D), _rows(1), _full2((D, 1)), _full2((1, 1))],
    out_specs=_rows(1),
    out_shape=jax.ShapeDtypeStruct((N, 1), f32),
)


def kernel(data, edge_index, W_in, b_in, W1, b1, W2, b2, W3, b3, W_out, b_out):
    src = edge_index[0]
    dst = edge_index[1]
    x_p = jnp.zeros((N_PAD, D), f32).at[:N].set(data)
    zeros2d = jnp.zeros((N_PAD, D), f32)
    zeros1d = jnp.zeros((N_PAD,), f32)
    ones_c = jnp.ones((CHUNK,), f32)
    bin2 = b_in.reshape(1, D)
    b1r = b1.reshape(1, D)
    b2r = b2.reshape(1, D)
    b3r = b3.reshape(1, D)
    boutr = b_out.reshape(1, 1)

    deg = _deg_kernel(dst, zeros1d, ones_c)
    d0 = deg[0].reshape(N_PAD, 1)
    d1 = deg[1].reshape(N_PAD, 1)
    g1, dinv = _k1_call(x_p, W_in, bin2, W1, b1r, d0, d1)
    s1 = _agg_kernel(g1, src, dst, zeros2d)
    g2 = _layer_call(s1, g1, W2, b2r, dinv)
    s2 = _agg_kernel(g2, src, dst, zeros2d)
    g3 = _layer_call(s2, g2, W3, b3r, dinv)
    s3 = _agg_kernel(g3, src, dst, zeros2d)
    return _head_call(s3, g3, dinv, W_out, boutr)


# speedup vs baseline: 1.0248x; 1.0248x over previous
import functools

import jax
import jax.numpy as jnp
from jax import lax
from jax.experimental import pallas as pl
from jax.experimental.pallas import tpu as pltpu
from jax.experimental.pallas import tpu_sc as plsc

N = 10000
E = 320000
D = 128
NC = 2
NS = 16
NW = NC * NS
CHUNK = 128
RING = 2
ISLOTS = 4
NCH = E // CHUNK
F = NCH // NW
EXTRA = NCH - NW * F
MAIN = F - 6
N_PAD = 10240
RPT = N_PAD // NS

_mesh = plsc.VectorSubcoreMesh(core_axis_name="core", subcore_axis_name="subcore")

f32 = jnp.float32


@functools.partial(
    pl.kernel,
    out_type=jax.ShapeDtypeStruct((NC, N_PAD), f32),
    mesh=_mesh,
    scratch_types=[
        pltpu.VMEM((ISLOTS, 2, CHUNK), jnp.int32),
        pltpu.VMEM((CHUNK,), f32),
        pltpu.VMEM_SHARED((N_PAD,), f32),
        pltpu.SemaphoreType.DMA((ISLOTS,)),
    ],
)
def _deg_kernel(ei_hbm, zeros_hbm, ones_hbm, deg_hbm,
                idx_v, ones_v, acc, dsems):
    cid = lax.axis_index("core")
    sid = lax.axis_index("subcore")
    wid = cid * NS + sid
    base = (wid * F + jnp.minimum(wid, EXTRA)) * CHUNK
    row0 = sid * RPT

    def _idx(k, j):
        return pltpu.make_async_copy(
            ei_hbm.at[pl.ds(0, 2), pl.ds(base + k * CHUNK, CHUNK)],
            idx_v.at[j], dsems.at[j])

    for j in range(ISLOTS):
        _idx(j, j).start()
    pltpu.sync_copy(zeros_hbm.at[pl.ds(row0, RPT)], acc.at[pl.ds(row0, RPT)])
    pltpu.sync_copy(ones_hbm, ones_v)
    plsc.subcore_barrier()

    @pl.loop(0, MAIN, step=ISLOTS)
    def _(kbase):
        for jj in range(ISLOTS):
            k = kbase + jj
            _idx(k, jj).wait()
            pltpu.sync_copy(ones_v, acc.at[idx_v.at[jj, 1]], add=True)
            _idx(k + ISLOTS, jj).start()

    for k in range(MAIN, F):
        jj = k % ISLOTS
        _idx(k, jj).wait()
        pltpu.sync_copy(ones_v, acc.at[idx_v.at[jj, 1]], add=True)
        if k + ISLOTS < F:
            _idx(k + ISLOTS, jj).start()

    @pl.when(wid < EXTRA)
    def _():
        pltpu.sync_copy(ei_hbm.at[pl.ds(0, 2), pl.ds(base + F * CHUNK, CHUNK)],
                        idx_v.at[0])
        pltpu.sync_copy(ones_v, acc.at[idx_v.at[0, 1]], add=True)

    plsc.subcore_barrier()
    pltpu.sync_copy(acc.at[pl.ds(row0, RPT)], deg_hbm.at[cid, pl.ds(row0, RPT)])


@functools.partial(
    pl.kernel,
    out_type=jax.ShapeDtypeStruct((NC, N_PAD, D), f32),
    mesh=_mesh,
    scratch_types=[
        pltpu.VMEM((ISLOTS, 2, CHUNK), jnp.int32),
        pltpu.VMEM((RING, CHUNK, D), f32),
        pltpu.VMEM_SHARED((N_PAD, D), f32),
        pltpu.SemaphoreType.DMA((RING,)),
        pltpu.SemaphoreType.DMA((ISLOTS,)),
    ],
)
def _agg_kernel(g_hbm, ei_hbm, zeros_hbm, s_hbm,
                idx_v, rows_v, acc, gsems, isems):
    cid = lax.axis_index("core")
    sid = lax.axis_index("subcore")
    wid = cid * NS + sid
    base = (wid * F + jnp.minimum(wid, EXTRA)) * CHUNK
    row0 = sid * RPT

    def _idx(k, j):
        return pltpu.make_async_copy(
            ei_hbm.at[pl.ds(0, 2), pl.ds(base + k * CHUNK, CHUNK)],
            idx_v.at[j], isems.at[j])

    def _gather(j, b):
        return pltpu.make_async_copy(g_hbm.at[idx_v.at[j, 0]], rows_v.at[b],
                                     gsems.at[b])

    for j in range(ISLOTS):
        _idx(j, j).start()
    pltpu.sync_copy(zeros_hbm.at[pl.ds(row0, RPT)], acc.at[pl.ds(row0, RPT)])
    plsc.subcore_barrier()

    for b in range(RING):
        _idx(b, b).wait()
        _gather(b, b).start()

    @pl.loop(0, MAIN, step=ISLOTS)
    def _(kbase):
        for jj in range(ISLOTS):
            k = kbase + jj
            b = jj % RING
            _gather(jj, b).wait()
            pltpu.sync_copy(rows_v.at[b], acc.at[idx_v.at[jj, 1]], add=True)
            _idx(k + ISLOTS, jj).start()
            j2 = (jj + RING) % ISLOTS
            _idx(k + RING, j2).wait()
            _gather(j2, b).start()

    for k in range(MAIN, F):
        jj = k % ISLOTS
        b = k % RING
        _gather(jj, b).wait()
        pltpu.sync_copy(rows_v.at[b], acc.at[idx_v.at[jj, 1]], add=True)
        if k + ISLOTS < F:
            _idx(k + ISLOTS, jj).start()
        if k + RING < F:
            j2 = (jj + RING) % ISLOTS
            _idx(k + RING, j2).wait()
            _gather(j2, b).start()

    @pl.when(wid < EXTRA)
    def _():
        pltpu.sync_copy(ei_hbm.at[pl.ds(0, 2), pl.ds(base + F * CHUNK, CHUNK)],
                        idx_v.at[0])
        pltpu.sync_copy(g_hbm.at[idx_v.at[0, 0]], rows_v.at[0])
        pltpu.sync_copy(rows_v.at[0], acc.at[idx_v.at[0, 1]], add=True)

    plsc.subcore_barrier()
    pltpu.sync_copy(acc.at[pl.ds(row0, RPT)], s_hbm.at[cid, pl.ds(row0, RPT)])


BR = 512
GRID = N_PAD // BR
_HI = lax.Precision.HIGHEST


def _mm(a, w):
    return jnp.dot(a, w, preferred_element_type=f32, precision=_HI)


def _k1_body(x_ref, win_ref, bin_ref, w1_ref, b1_ref, d0_ref, d1_ref,
             g_ref, dinv_ref):
    dinv = lax.rsqrt(d0_ref[...] + d1_ref[...] + 1.0)
    h = jnp.maximum(_mm(x_ref[...], win_ref[...]) + bin_ref[...], 0.0)
    g_ref[...] = (_mm(h, w1_ref[...]) + b1_ref[...]) * dinv
    dinv_ref[...] = dinv


def _layer_body(s_ref, g_ref, w_ref, b_ref, dinv_ref, out_ref):
    dinv = dinv_ref[...]
    x = jnp.maximum((s_ref[0] + s_ref[1] + g_ref[...]) * dinv, 0.0)
    out_ref[...] = (_mm(x, w_ref[...]) + b_ref[...]) * dinv


def _head_body(s_ref, g_ref, dinv_ref, wout_ref, bout_ref, out_ref):
    x = jnp.maximum((s_ref[0] + s_ref[1] + g_ref[...]) * dinv_ref[...], 0.0)
    out_ref[...] = _mm(x, wout_ref[...]) + bout_ref[...]


_full2 = lambda shape: pl.BlockSpec(shape, lambda i: (0, 0))
_rows = lambda w: pl.BlockSpec((BR, w), lambda i: (i, 0))
_srow = pl.BlockSpec((2, BR, D), lambda i: (0, i, 0))

_k1_call = pl.pallas_call(
    _k1_body,
    grid=(GRID,),
    in_specs=[_rows(D), _full2((D, D)), _full2((1, D)), _full2((D, D)),
              _full2((1, D)), _rows(1), _rows(1)],
    out_specs=[_rows(D), _rows(1)],
    out_shape=[jax.ShapeDtypeStruct((N_PAD, D), f32),
               jax.ShapeDtypeStruct((N_PAD, 1), f32)],
)

_layer_call = pl.pallas_call(
    _layer_body,
    grid=(GRID,),
    in_specs=[_srow, _rows(D), _full2((D, D)), _full2((1, D)), _rows(1)],
    out_specs=_rows(D),
    out_shape=jax.ShapeDtypeStruct((N_PAD, D), f32),
)

_head_call = pl.pallas_call(
    _head_body,
    grid=(GRID,),
    in_specs=[_srow, _rows(D), _rows(1), _full2((D, 1)), _full2((1, 1))],
    out_specs=_rows(1),
    out_shape=jax.ShapeDtypeStruct((N, 1), f32),
)


def kernel(data, edge_index, W_in, b_in, W1, b1, W2, b2, W3, b3, W_out, b_out):
    x_p = jnp.zeros((N_PAD, D), f32).at[:N].set(data)
    zeros2d = jnp.zeros((N_PAD, D), f32)
    zeros1d = jnp.zeros((N_PAD,), f32)
    ones_c = jnp.ones((CHUNK,), f32)
    bin2 = b_in.reshape(1, D)
    b1r = b1.reshape(1, D)
    b2r = b2.reshape(1, D)
    b3r = b3.reshape(1, D)
    boutr = b_out.reshape(1, 1)

    deg = _deg_kernel(edge_index, zeros1d, ones_c)
    d0 = deg[0].reshape(N_PAD, 1)
    d1 = deg[1].reshape(N_PAD, 1)
    g1, dinv = _k1_call(x_p, W_in, bin2, W1, b1r, d0, d1)
    s1 = _agg_kernel(g1, edge_index, zeros2d)
    g2 = _layer_call(s1, g1, W2, b2r, dinv)
    s2 = _agg_kernel(g2, edge_index, zeros2d)
    g3 = _layer_call(s2, g2, W3, b3r, dinv)
    s3 = _agg_kernel(g3, edge_index, zeros2d)
    return _head_call(s3, g3, dinv, W_out, boutr)


# speedup vs baseline: 1.1120x; 1.0851x over previous
import functools

import jax
import jax.numpy as jnp
from jax import lax
from jax.experimental import pallas as pl
from jax.experimental.pallas import tpu as pltpu
from jax.experimental.pallas import tpu_sc as plsc

N = 10000
E = 320000
D = 128
NC = 2
NS = 16
NW = NC * NS
CHUNK = 128
RING = 2
ISLOTS = 4
NCH = E // CHUNK
F = NCH // NW
EXTRA = NCH - NW * F
MAIN = F - 6
N_PAD = 10240
RPT = N_PAD // NS

_mesh = plsc.VectorSubcoreMesh(core_axis_name="core", subcore_axis_name="subcore")

f32 = jnp.float32


@functools.partial(
    pl.kernel,
    out_type=jax.ShapeDtypeStruct((NC, N_PAD), f32),
    mesh=_mesh,
    scratch_types=[
        pltpu.VMEM((ISLOTS, 2, CHUNK), jnp.int32),
        pltpu.VMEM((CHUNK,), f32),
        pltpu.VMEM_SHARED((N_PAD,), f32),
        pltpu.SemaphoreType.DMA((ISLOTS,)),
    ],
)
def _deg_kernel(ei_hbm, zeros_hbm, ones_hbm, deg_hbm,
                idx_v, ones_v, acc, dsems):
    cid = lax.axis_index("core")
    sid = lax.axis_index("subcore")
    wid = cid * NS + sid
    base = (wid * F + jnp.minimum(wid, EXTRA)) * CHUNK
    row0 = sid * RPT

    def _idx(k, j):
        return pltpu.make_async_copy(
            ei_hbm.at[pl.ds(0, 2), pl.ds(base + k * CHUNK, CHUNK)],
            idx_v.at[j], dsems.at[j])

    for j in range(ISLOTS):
        _idx(j, j).start()
    pltpu.sync_copy(zeros_hbm.at[pl.ds(row0, RPT)], acc.at[pl.ds(row0, RPT)])
    pltpu.sync_copy(ones_hbm, ones_v)
    plsc.subcore_barrier()

    @pl.loop(0, MAIN, step=ISLOTS)
    def _(kbase):
        for jj in range(ISLOTS):
            k = kbase + jj
            _idx(k, jj).wait()
            pltpu.sync_copy(ones_v, acc.at[idx_v.at[jj, 1]], add=True)
            _idx(k + ISLOTS, jj).start()

    for k in range(MAIN, F):
        jj = k % ISLOTS
        _idx(k, jj).wait()
        pltpu.sync_copy(ones_v, acc.at[idx_v.at[jj, 1]], add=True)
        if k + ISLOTS < F:
            _idx(k + ISLOTS, jj).start()

    @pl.when(wid < EXTRA)
    def _():
        pltpu.sync_copy(ei_hbm.at[pl.ds(0, 2), pl.ds(base + F * CHUNK, CHUNK)],
                        idx_v.at[0])
        pltpu.sync_copy(ones_v, acc.at[idx_v.at[0, 1]], add=True)

    plsc.subcore_barrier()
    pltpu.sync_copy(acc.at[pl.ds(row0, RPT)], deg_hbm.at[cid, pl.ds(row0, RPT)])


@functools.partial(
    pl.kernel,
    out_type=jax.ShapeDtypeStruct((NC, N_PAD, D), f32),
    mesh=_mesh,
    scratch_types=[
        pltpu.VMEM((ISLOTS, 2, CHUNK), jnp.int32),
        pltpu.VMEM((RING, CHUNK, D), f32),
        pltpu.VMEM_SHARED((N_PAD, D), f32),
        pltpu.SemaphoreType.DMA((RING,)),
        pltpu.SemaphoreType.DMA((ISLOTS,)),
    ],
)
def _agg_kernel(g_hbm, ei_hbm, zeros_hbm, s_hbm,
                idx_v, rows_v, acc, gsems, isems):
    cid = lax.axis_index("core")
    sid = lax.axis_index("subcore")
    wid = cid * NS + sid
    base = (wid * F + jnp.minimum(wid, EXTRA)) * CHUNK
    row0 = sid * RPT

    def _idx(k, j):
        return pltpu.make_async_copy(
            ei_hbm.at[pl.ds(0, 2), pl.ds(base + k * CHUNK, CHUNK)],
            idx_v.at[j], isems.at[j])

    def _gather(j, b):
        return pltpu.make_async_copy(g_hbm.at[idx_v.at[j, 0]], rows_v.at[b],
                                     gsems.at[b])

    for j in range(ISLOTS):
        _idx(j, j).start()
    pltpu.sync_copy(zeros_hbm.at[pl.ds(row0, RPT)], acc.at[pl.ds(row0, RPT)])
    plsc.subcore_barrier()

    for b in range(RING):
        _idx(b, b).wait()
        _gather(b, b).start()

    @pl.loop(0, MAIN, step=ISLOTS)
    def _(kbase):
        for jj in range(ISLOTS):
            k = kbase + jj
            b = jj % RING
            _gather(jj, b).wait()
            pltpu.sync_copy(rows_v.at[b], acc.at[idx_v.at[jj, 1]], add=True)
            _idx(k + ISLOTS, jj).start()
            j2 = (jj + RING) % ISLOTS
            _idx(k + RING, j2).wait()
            _gather(j2, b).start()

    for k in range(MAIN, F):
        jj = k % ISLOTS
        b = k % RING
        _gather(jj, b).wait()
        pltpu.sync_copy(rows_v.at[b], acc.at[idx_v.at[jj, 1]], add=True)
        if k + ISLOTS < F:
            _idx(k + ISLOTS, jj).start()
        if k + RING < F:
            j2 = (jj + RING) % ISLOTS
            _idx(k + RING, j2).wait()
            _gather(j2, b).start()

    @pl.when(wid < EXTRA)
    def _():
        pltpu.sync_copy(ei_hbm.at[pl.ds(0, 2), pl.ds(base + F * CHUNK, CHUNK)],
                        idx_v.at[0])
        pltpu.sync_copy(g_hbm.at[idx_v.at[0, 0]], rows_v.at[0])
        pltpu.sync_copy(rows_v.at[0], acc.at[idx_v.at[0, 1]], add=True)

    plsc.subcore_barrier()
    pltpu.sync_copy(acc.at[pl.ds(row0, RPT)], s_hbm.at[cid, pl.ds(row0, RPT)])


BR = 1024
GRID = N_PAD // BR


def _mm(a, w):
    return jnp.dot(a, w, preferred_element_type=f32)


def _k1_body(x_ref, win_ref, bin_ref, w1_ref, b1_ref, d0_ref, d1_ref,
             g_ref, dinv_ref):
    dinv = lax.rsqrt(d0_ref[...] + d1_ref[...] + 1.0)
    h = jnp.maximum(_mm(x_ref[...], win_ref[...]) + bin_ref[...], 0.0)
    g_ref[...] = (_mm(h, w1_ref[...]) + b1_ref[...]) * dinv
    dinv_ref[...] = dinv


def _layer_body(s_ref, g_ref, w_ref, b_ref, dinv_ref, out_ref):
    dinv = dinv_ref[...]
    x = jnp.maximum((s_ref[0] + s_ref[1] + g_ref[...]) * dinv, 0.0)
    out_ref[...] = (_mm(x, w_ref[...]) + b_ref[...]) * dinv


def _head_body(s_ref, g_ref, dinv_ref, wout_ref, bout_ref, out_ref):
    x = jnp.maximum((s_ref[0] + s_ref[1] + g_ref[...]) * dinv_ref[...], 0.0)
    out_ref[...] = _mm(x, wout_ref[...]) + bout_ref[...]


_full2 = lambda shape: pl.BlockSpec(shape, lambda i: (0, 0))
_rows = lambda w: pl.BlockSpec((BR, w), lambda i: (i, 0))
_srow = pl.BlockSpec((2, BR, D), lambda i: (0, i, 0))

_k1_call = pl.pallas_call(
    _k1_body,
    grid=(GRID,),
    in_specs=[_rows(D), _full2((D, D)), _full2((1, D)), _full2((D, D)),
              _full2((1, D)), _rows(1), _rows(1)],
    out_specs=[_rows(D), _rows(1)],
    out_shape=[jax.ShapeDtypeStruct((N_PAD, D), f32),
               jax.ShapeDtypeStruct((N_PAD, 1), f32)],
)

_layer_call = pl.pallas_call(
    _layer_body,
    grid=(GRID,),
    in_specs=[_srow, _rows(D), _full2((D, D)), _full2((1, D)), _rows(1)],
    out_specs=_rows(D),
    out_shape=jax.ShapeDtypeStruct((N_PAD, D), f32),
)

_head_call = pl.pallas_call(
    _head_body,
    grid=(GRID,),
    in_specs=[_srow, _rows(D), _rows(1), _full2((D, 1)), _full2((1, 1))],
    out_specs=_rows(1),
    out_shape=jax.ShapeDtypeStruct((N, 1), f32),
)


def kernel(data, edge_index, W_in, b_in, W1, b1, W2, b2, W3, b3, W_out, b_out):
    x_p = jnp.zeros((N_PAD, D), f32).at[:N].set(data)
    zeros2d = jnp.zeros((N_PAD, D), f32)
    zeros1d = jnp.zeros((N_PAD,), f32)
    ones_c = jnp.ones((CHUNK,), f32)
    bin2 = b_in.reshape(1, D)
    b1r = b1.reshape(1, D)
    b2r = b2.reshape(1, D)
    b3r = b3.reshape(1, D)
    boutr = b_out.reshape(1, 1)

    deg = _deg_kernel(edge_index, zeros1d, ones_c)
    d0 = deg[0].reshape(N_PAD, 1)
    d1 = deg[1].reshape(N_PAD, 1)
    g1, dinv = _k1_call(x_p, W_in, bin2, W1, b1r, d0, d1)
    s1 = _agg_kernel(g1, edge_index, zeros2d)
    g2 = _layer_call(s1, g1, W2, b2r, dinv)
    s2 = _agg_kernel(g2, edge_index, zeros2d)
    g3 = _layer_call(s2, g2, W3, b3r, dinv)
    s3 = _agg_kernel(g3, edge_index, zeros2d)
    return _head_call(s3, g3, dinv, W_out, boutr)
